# baseline (speedup 1.0000x reference)
import functools

import jax
import jax.numpy as jnp
from jax import lax
from jax.experimental import pallas as pl
from jax.experimental.pallas import tpu as pltpu
from jax.experimental.pallas import tpu_sc as plsc

EMBED = 64

_info = plsc.get_sparse_core_info()
_NC, _NS = _info.num_cores, _info.num_subcores
_NW = _NC * _NS

_STREAM = 128
_NT = 1024 * 200
_B_PER_W = _NT // _NW
_NCH = _B_PER_W // _STREAM

_mesh = plsc.VectorSubcoreMesh(core_axis_name="c", subcore_axis_name="s")


@functools.partial(
    pl.kernel,
    mesh=_mesh,
    out_type=jax.ShapeDtypeStruct((_NT, EMBED), jnp.float32),
    compiler_params=pltpu.CompilerParams(use_tc_tiling_on_sc=False),
    scratch_types=[
        pltpu.VMEM((_NCH, _STREAM), jnp.int32),
        pltpu.VMEM((_STREAM, EMBED), jnp.float32),
        pltpu.SemaphoreType.DMA,
    ],
)
def _gather(idx_hbm, table_hbm, out_hbm, idx_v, rows_v, gsem):
    wid = lax.axis_index("s") * _NC + lax.axis_index("c")
    base = wid * _B_PER_W
    pltpu.sync_copy(idx_hbm.at[wid], idx_v)

    def body(g, carry):
        pltpu.async_copy(table_hbm.at[idx_v.at[g]], rows_v, gsem).wait()
        pltpu.sync_copy(rows_v, out_hbm.at[pl.ds(base + g * _STREAM, _STREAM)])
        return carry

    lax.fori_loop(0, _NCH, body, 0)


def kernel(inputtokens, table):
    idx = inputtokens.reshape(_NW, _NCH, _STREAM).astype(jnp.int32)
    out = _gather(idx, table)
    return out.reshape(inputtokens.shape + (EMBED,))


# speedup vs baseline: 1.0409x; 1.0409x over previous
import functools

import jax
import jax.numpy as jnp
from jax import lax
from jax.experimental import pallas as pl
from jax.experimental.pallas import tpu as pltpu
from jax.experimental.pallas import tpu_sc as plsc

EMBED = 64

_info = plsc.get_sparse_core_info()
_NC, _NS = _info.num_cores, _info.num_subcores
_NW = _NC * _NS

_STREAM = 128
_SUB = 5
_CHUNK = _STREAM * _SUB
_NT = 1024 * 200
_B_PER_W = _NT // _NW
_NSTR = _B_PER_W // _STREAM
_NCHUNK = _B_PER_W // _CHUNK

_mesh = plsc.VectorSubcoreMesh(core_axis_name="c", subcore_axis_name="s")


@functools.partial(
    pl.kernel,
    mesh=_mesh,
    out_type=jax.ShapeDtypeStruct((_NT, EMBED), jnp.float32),
    compiler_params=pltpu.CompilerParams(use_tc_tiling_on_sc=False),
    scratch_types=[
        pltpu.VMEM((_NSTR, _STREAM), jnp.int32),
        pltpu.VMEM((_CHUNK, EMBED), jnp.float32),
        pltpu.VMEM((_CHUNK, EMBED), jnp.float32),
        pltpu.SemaphoreType.DMA,
        pltpu.SemaphoreType.DMA,
        pltpu.SemaphoreType.DMA,
        pltpu.SemaphoreType.DMA,
    ],
)
def _gather(idx_hbm, table_hbm, out_hbm, idx_v, rows0, rows1,
            gs0, gs1, os0, os1):
    wid = lax.axis_index("s") * _NC + lax.axis_index("c")
    base = wid * _B_PER_W
    bufs = ((rows0, gs0, os0), (rows1, gs1, os1))
    pltpu.sync_copy(idx_hbm.at[wid], idx_v)

    def fire_gathers(c, b):
        rows, gs, _ = bufs[b]
        for s in range(_SUB):
            pltpu.async_copy(table_hbm.at[idx_v.at[c * _SUB + s]],
                             rows.at[pl.ds(s * _STREAM, _STREAM)], gs)

    def drain_gathers(b):
        rows, gs, _ = bufs[b]
        pltpu.make_async_copy(table_hbm.at[pl.ds(0, _CHUNK)], rows, gs).wait()

    def out_copy(c, b):
        rows, _, os = bufs[b]
        return pltpu.async_copy(
            rows, out_hbm.at[pl.ds(base + c * _CHUNK, _CHUNK)], os)

    fire_gathers(0, 0)
    fire_gathers(1, 1)

    def body(j, carry):
        c0 = 2 * j
        c1 = c0 + 1
        drain_gathers(0)
        o0 = out_copy(c0, 0)
        drain_gathers(1)
        o1 = out_copy(c1, 1)
        o0.wait()
        fire_gathers(c0 + 2, 0)
        o1.wait()
        fire_gathers(c1 + 2, 1)
        return carry

    lax.fori_loop(0, _NCHUNK // 2 - 1, body, 0)

    drain_gathers(0)
    o0 = out_copy(_NCHUNK - 2, 0)
    drain_gathers(1)
    o1 = out_copy(_NCHUNK - 1, 1)
    o0.wait()
    o1.wait()


def kernel(inputtokens, table):
    idx = inputtokens.reshape(_NW, _NSTR, _STREAM).astype(jnp.int32)
    out = _gather(idx, table)
    return out.reshape(inputtokens.shape + (EMBED,))
